# Initial kernel scaffold; baseline (speedup 1.0000x reference)
import jax
import jax.numpy as jnp
from jax.experimental import pallas as pl


def kernel(x_g1, pos_g1, batch_g1, x_g2, pos_g2, batch_g2, W_emb, b_emb, W_att, b_att, W_fc, b_fc):
    raise NotImplementedError("write your pallas kernel here")


# speedup vs baseline: 2.1406x; 2.1406x over previous
import functools

import jax
import jax.numpy as jnp
from jax import lax
from jax.experimental import pallas as pl
from jax.experimental.pallas import tpu as pltpu
from jax.experimental.pallas import tpu_sc as plsc

_HIDDEN = 256
_OUT = 2
_NPATCH = 6
_PS = 202.0
_B = 16
_N = 100000
_D = 128
_NSEG = _NPATCH * _NPATCH * _B
_SEGP = 640
_ROWS_PER_TILE = _SEGP // 16

_CGRID = 2048
_NC = 2
_NS = 16
_NW = _NC * _NS
_PER_W = 3200
_NPAD = _NW * _PER_W
_CH = 128
_NCHUNK = _PER_W // _CH


def _gather16(v, idx):
    return lax.gather(
        v,
        idx[:, None],
        dimension_numbers=lax.GatherDimensionNumbers(
            offset_dims=(), collapsed_slice_dims=(0,), start_index_map=(0,)),
        slice_sizes=(1,),
        mode=lax.GatherScatterMode.PROMISE_IN_BOUNDS,
    )


def _sc_body(x1, p1, b1, x2, p2, b2, sums, cnts,
             xbuf, posbuf, batbuf, idxbuf, onesb, zx,
             ax1, ax2, ac1, ac2):
    c = lax.axis_index("c")
    s = lax.axis_index("s")

    z16 = jnp.zeros((16,), jnp.float32)
    o16 = jnp.ones((16,), jnp.float32)
    lanes = lax.iota(jnp.int32, 16)

    def zrow(i, carry):
        for j in range(8):
            zx[i, pl.ds(j * 16, 16)] = z16
        return carry

    lax.fori_loop(0, _ROWS_PER_TILE, zrow, 0)

    def orow(i, carry):
        for j in range(8):
            onesb[i, pl.ds(j * 16, 16)] = o16
        return carry

    lax.fori_loop(0, _CH, orow, 0)

    r0 = s * _ROWS_PER_TILE
    for ax in (ax1, ax2, ac1, ac2):
        pltpu.sync_copy(zx, ax.at[pl.ds(r0, _ROWS_PER_TILE)])

    plsc.subcore_barrier()

    eidx = (lanes & 7) * 2
    oidx = eidx + 1
    lo8 = lanes < 8

    base = (c * _NS + s) * _PER_W

    def do_graph(xref, pref, bref, accx, accc):
        def chunk(k, carry):
            row0 = base + k * _CH
            pltpu.sync_copy(xref.at[pl.ds(row0, _CH)], xbuf)
            pltpu.sync_copy(pref.at[pl.ds(2 * row0, 2 * _CH)], posbuf)
            pltpu.sync_copy(bref.at[pl.ds(row0, _CH)], batbuf)

            def seg16(i, carry2):
                va = posbuf[pl.ds(32 * i, 16)]
                vb = posbuf[pl.ds(32 * i + 16, 16)]
                ta = jnp.clip((va / _PS).astype(jnp.int32), 0, _NPATCH - 1)
                tb = jnp.clip((vb / _PS).astype(jnp.int32), 0, _NPATCH - 1)
                px = jnp.where(lo8, _gather16(ta, eidx), _gather16(tb, eidx))
                py = jnp.where(lo8, _gather16(ta, oidx), _gather16(tb, oidx))
                bt = batbuf[pl.ds(16 * i, 16)]
                seg = jnp.minimum(px * (_NPATCH * _B) + py * _B + bt, _NSEG)
                idxbuf[pl.ds(16 * i, 16)] = seg
                return carry2

            lax.fori_loop(0, _CH // 16, seg16, 0)
            pltpu.sync_copy(xbuf, accx.at[idxbuf], add=True)
            pltpu.sync_copy(onesb, accc.at[idxbuf], add=True)
            return carry

        lax.fori_loop(0, _NCHUNK, chunk, 0)

    do_graph(x1, p1, b1, ax1, ac1)
    do_graph(x2, p2, b2, ax2, ac2)
    plsc.subcore_barrier()

    for g, ax in enumerate((ax1, ax2)):
        pltpu.sync_copy(ax.at[pl.ds(r0, _ROWS_PER_TILE)],
                        sums.at[c, g, pl.ds(r0, _ROWS_PER_TILE)])
    for g, ac in enumerate((ac1, ac2)):
        pltpu.sync_copy(ac.at[pl.ds(r0, _ROWS_PER_TILE)],
                        cnts.at[c, g, pl.ds(r0, _ROWS_PER_TILE)])


_sc_segment_sums = functools.partial(
    pl.kernel,
    out_type=(
        jax.ShapeDtypeStruct((_NC, 2, _SEGP, _D), jnp.float32),
        jax.ShapeDtypeStruct((_NC, 2, _SEGP, _D), jnp.float32),
    ),
    mesh=plsc.VectorSubcoreMesh(core_axis_name="c", subcore_axis_name="s"),
    scratch_types=[
        pltpu.VMEM((_CH, _D), jnp.float32),
        pltpu.VMEM((2 * _CH,), jnp.float32),
        pltpu.VMEM((_CH,), jnp.int32),
        pltpu.VMEM((_CH,), jnp.int32),
        pltpu.VMEM((_CH, _D), jnp.float32),
        pltpu.VMEM((_ROWS_PER_TILE, _D), jnp.float32),
        pltpu.VMEM_SHARED((_SEGP, _D), jnp.float32),
        pltpu.VMEM_SHARED((_SEGP, _D), jnp.float32),
        pltpu.VMEM_SHARED((_SEGP, _D), jnp.float32),
        pltpu.VMEM_SHARED((_SEGP, _D), jnp.float32),
    ],
)(_sc_body)


def _head_body(sums_ref, cnts_ref, wemb_ref, bemb_ref, watt_ref, batt_ref,
               wfc_ref, bfc_ref, out_ref):
    s1 = sums_ref[0, 0] + sums_ref[1, 0]
    s2 = sums_ref[0, 1] + sums_ref[1, 1]
    c1 = cnts_ref[0, 0, :, 0] + cnts_ref[1, 0, :, 0]
    c2 = cnts_ref[0, 1, :, 0] + cnts_ref[1, 1, :, 0]
    m1 = (s1 / jnp.maximum(c1, 1.0)[:, None])[:_NSEG]
    m2 = (s2 / jnp.maximum(c2, 1.0)[:, None])[:_NSEG]
    emb = (jnp.dot(m1, wemb_ref[:_D, :], preferred_element_type=jnp.float32)
           + jnp.dot(m2, wemb_ref[_D:, :], preferred_element_type=jnp.float32)
           + bemb_ref[:][None, :])
    logits = jnp.dot(emb, watt_ref[:], preferred_element_type=jnp.float32)
    logits = logits[:, 0] + batt_ref[0]
    lg = logits.reshape(_NPATCH * _NPATCH, _B)
    lg = lg - jnp.max(lg, axis=0, keepdims=True)
    e = jnp.exp(lg)
    w = e / jnp.sum(e, axis=0, keepdims=True)
    e3 = emb.reshape(_NPATCH * _NPATCH, _B, _HIDDEN)
    attended = jnp.sum(w[:, :, None] * e3, axis=0)
    out_ref[...] = (jnp.dot(attended, wfc_ref[:],
                            preferred_element_type=jnp.float32)
                    + bfc_ref[:][None, :])


def kernel(x_g1, pos_g1, batch_g1, x_g2, pos_g2, batch_g2,
           W_emb, b_emb, W_att, b_att, W_fc, b_fc):
    pad = _NPAD - _N
    padf = jnp.zeros((pad, _D), jnp.float32)
    padp = jnp.zeros((pad * 2,), jnp.float32)
    padb = jnp.full((pad,), jnp.int32(1 << 20))
    x1 = jnp.concatenate([x_g1, padf], axis=0)
    x2 = jnp.concatenate([x_g2, padf], axis=0)
    p1 = jnp.concatenate([pos_g1.reshape(-1), padp])
    p2 = jnp.concatenate([pos_g2.reshape(-1), padp])
    b1 = jnp.concatenate([batch_g1.astype(jnp.int32), padb])
    b2 = jnp.concatenate([batch_g2.astype(jnp.int32), padb])

    sums, cnts = _sc_segment_sums(x1, p1, b1, x2, p2, b2)

    return pl.pallas_call(
        _head_body,
        out_shape=jax.ShapeDtypeStruct((_B, _OUT), jnp.float32),
    )(sums, cnts, W_emb, b_emb, W_att, b_att, W_fc, b_fc)
